# Initial kernel scaffold; baseline (speedup 1.0000x reference)
import jax
import jax.numpy as jnp
from jax.experimental import pallas as pl


def kernel(x, edge_index, W1, b1, W2, b2):
    raise NotImplementedError("write your pallas kernel here")


# speedup vs baseline: 27.7714x; 27.7714x over previous
import functools

import jax
import jax.numpy as jnp
from jax import lax
from jax.experimental import pallas as pl
from jax.experimental.pallas import tpu as pltpu
from jax.experimental.pallas import tpu_sc as plsc

N = 10000
E = 320000
D = 128

NC = 2
NS = 16
NW = NC * NS
EPW = E // NW
K = 100
NCHUNK = EPW // K
RPT = 624
ZR = 16
TAIL = N - NS * RPT

_mesh = plsc.VectorSubcoreMesh(core_axis_name="c", subcore_axis_name="s")


def _fill(buf, rows, value):
    vec = jnp.full((16,), value, jnp.float32)

    def body(i, carry):
        buf[i // (D // 16), pl.ds((i % (D // 16)) * 16, 16)] = vec
        return carry

    lax.fori_loop(0, rows * (D // 16), body, 0)


def _zero_acc(zbuf, acc, s):
    _fill(zbuf, ZR, 0.0)

    def body(i, carry):
        pltpu.sync_copy(zbuf, acc.at[pl.ds(s * RPT + i * ZR, ZR)])
        return carry

    lax.fori_loop(0, RPT // ZR, body, 0)

    @pl.when(s == NS - 1)
    def _():
        pltpu.sync_copy(zbuf.at[pl.ds(0, TAIL)], acc.at[pl.ds(NS * RPT, TAIL)])


def _copy_out(acc, out_hbm, c, s):
    pltpu.sync_copy(acc.at[pl.ds(s * RPT, RPT)],
                    out_hbm.at[c, pl.ds(s * RPT, RPT)])

    @pl.when(s == NS - 1)
    def _():
        pltpu.sync_copy(acc.at[pl.ds(NS * RPT, TAIL)],
                        out_hbm.at[c, pl.ds(NS * RPT, TAIL)])


@functools.partial(
    pl.kernel,
    out_type=jax.ShapeDtypeStruct((NC, N, D), jnp.float32),
    mesh=_mesh,
    scratch_types=[
        pltpu.VMEM((NCHUNK, K), jnp.int32),
        pltpu.VMEM((K, D), jnp.float32),
        pltpu.VMEM((ZR, D), jnp.float32),
        pltpu.VMEM_SHARED((N, D), jnp.float32),
    ],
)
def _deg_kernel(dst_hbm, out_hbm, dst_v, ones_v, zbuf, acc):
    c = lax.axis_index("c")
    s = lax.axis_index("s")
    wid = s * NC + c
    _zero_acc(zbuf, acc, s)
    _fill(ones_v, K, 1.0)
    pltpu.sync_copy(dst_hbm.at[wid], dst_v)
    plsc.subcore_barrier()

    def body(j, carry):
        pltpu.sync_copy(ones_v, acc.at[dst_v.at[j]], add=True)
        return carry

    lax.fori_loop(0, NCHUNK, body, 0)
    plsc.subcore_barrier()
    _copy_out(acc, out_hbm, c, s)


@functools.partial(
    pl.kernel,
    out_type=jax.ShapeDtypeStruct((NC, N, D), jnp.float32),
    mesh=_mesh,
    scratch_types=[
        pltpu.VMEM((4, K), jnp.int32),
        pltpu.VMEM((4, K), jnp.int32),
        pltpu.VMEM((K, D), jnp.float32),
        pltpu.VMEM((K, D), jnp.float32),
        pltpu.VMEM((ZR, D), jnp.float32),
        pltpu.VMEM_SHARED((N, D), jnp.float32),
        pltpu.SemaphoreType.DMA,
        pltpu.SemaphoreType.DMA,
        pltpu.SemaphoreType.DMA,
        pltpu.SemaphoreType.DMA,
        pltpu.SemaphoreType.DMA,
        pltpu.SemaphoreType.DMA,
    ],
)
def _edge_kernel(y_hbm, src_hbm, dst_hbm, out_hbm,
                 sidx, didx, buf0, buf1, zbuf, acc,
                 sem0, sem1, isem0, isem1, isem2, isem3):
    c = lax.axis_index("c")
    s = lax.axis_index("s")
    wid = s * NC + c
    bufs = (buf0, buf1)
    sems = (sem0, sem1)
    isems = (isem0, isem1, isem2, isem3)

    def load_idx(chunk, row):
        pltpu.async_copy(src_hbm.at[wid, chunk], sidx.at[row], isems[row])
        pltpu.async_copy(dst_hbm.at[wid, chunk], didx.at[row], isems[row])

    def wait_idx(chunk, row):
        pltpu.make_async_copy(src_hbm.at[wid, chunk], sidx.at[row],
                              isems[row]).wait()
        pltpu.make_async_copy(dst_hbm.at[wid, chunk], didx.at[row],
                              isems[row]).wait()

    _zero_acc(zbuf, acc, s)
    for r in range(4):
        load_idx(r, r)
    plsc.subcore_barrier()
    for r in range(2):
        wait_idx(r, r)
        pltpu.async_copy(y_hbm.at[sidx.at[r]], bufs[r], sems[r])

    def body(jj, carry):
        for b in range(4):
            j = 4 * jj + b
            p = b % 2
            pltpu.make_async_copy(y_hbm.at[sidx.at[b]], bufs[p],
                                  sems[p]).wait()
            pltpu.sync_copy(bufs[p], acc.at[didx.at[b]], add=True)

            @pl.when(j + 4 < NCHUNK)
            def _():
                load_idx(j + 4, b)

            @pl.when(j + 2 < NCHUNK)
            def _():
                wait_idx(j + 2, (b + 2) % 4)
                pltpu.async_copy(y_hbm.at[sidx.at[(b + 2) % 4]], bufs[p],
                                 sems[p])
        return carry

    lax.fori_loop(0, NCHUNK // 4, body, 0)
    plsc.subcore_barrier()
    _copy_out(acc, out_hbm, c, s)


R = 1000


def _tc1_body(degp_ref, x_ref, w_ref, y_ref, dis_ref):
    deg = degp_ref[0] + degp_ref[1] + 1.0
    dis = lax.rsqrt(deg)
    dis_ref[...] = dis
    y_ref[...] = dis * jnp.dot(x_ref[...], w_ref[...],
                               preferred_element_type=jnp.float32)


def _tc2_body(aggp_ref, y1_ref, dis_ref, b1_ref, w_ref, y2_ref):
    agg = aggp_ref[0] + aggp_ref[1] + y1_ref[...]
    h = jnp.maximum(dis_ref[...] * agg + b1_ref[...], 0.0)
    y2_ref[...] = dis_ref[...] * jnp.dot(h, w_ref[...],
                                         preferred_element_type=jnp.float32)


def _tc3_body(aggp_ref, y2_ref, dis_ref, b2_ref, z_ref):
    z_ref[...] = (dis_ref[...] * (aggp_ref[0] + aggp_ref[1] + y2_ref[...])
                  + b2_ref[...])


_part_spec = pl.BlockSpec((NC, R, D), lambda i: (0, i, 0))
_row_spec = pl.BlockSpec((R, D), lambda i: (i, 0))
_mat_spec = pl.BlockSpec((D, D), lambda i: (0, 0))
_vec_spec = pl.BlockSpec((1, D), lambda i: (0, 0))
_rows = jax.ShapeDtypeStruct((N, D), jnp.float32)

_tc1 = pl.pallas_call(
    _tc1_body, grid=(N // R,),
    in_specs=[_part_spec, _row_spec, _mat_spec],
    out_specs=[_row_spec, _row_spec],
    out_shape=[_rows, _rows],
)

_tc2 = pl.pallas_call(
    _tc2_body, grid=(N // R,),
    in_specs=[_part_spec, _row_spec, _row_spec, _vec_spec, _mat_spec],
    out_specs=_row_spec,
    out_shape=_rows,
)

_tc3 = pl.pallas_call(
    _tc3_body, grid=(N // R,),
    in_specs=[_part_spec, _row_spec, _row_spec, _vec_spec],
    out_specs=_row_spec,
    out_shape=_rows,
)


def kernel(x, edge_index, W1, b1, W2, b2):
    src3 = edge_index[0].reshape(NW, NCHUNK, K)
    dst3 = edge_index[1].reshape(NW, NCHUNK, K)
    degp = _deg_kernel(dst3)
    y1, dis = _tc1(degp, x, W1)
    agg1 = _edge_kernel(y1, src3, dst3)
    y2 = _tc2(agg1, y1, dis, b1.reshape(1, D), W2)
    agg2 = _edge_kernel(y2, src3, dst3)
    z = _tc3(agg2, y2, dis, b2.reshape(1, D))
    return z


# speedup vs baseline: 31.5298x; 1.1353x over previous
import functools

import jax
import jax.numpy as jnp
from jax import lax
from jax.experimental import pallas as pl
from jax.experimental.pallas import tpu as pltpu
from jax.experimental.pallas import tpu_sc as plsc

N = 10000
E = 320000
D = 128

NC = 2
NS = 16
NW = NC * NS
EPW = E // NW
K = 100
NCHUNK = EPW // K
RPT = 624
ZR = 16
TAIL = N - NS * RPT

_mesh = plsc.VectorSubcoreMesh(core_axis_name="c", subcore_axis_name="s")


def _fill(buf, rows, value):
    vec = jnp.full((16,), value, jnp.float32)

    def body(i, carry):
        buf[i // (D // 16), pl.ds((i % (D // 16)) * 16, 16)] = vec
        return carry

    lax.fori_loop(0, rows * (D // 16), body, 0)


def _zero_acc(zbuf, acc, s):
    _fill(zbuf, ZR, 0.0)

    def body(i, carry):
        pltpu.sync_copy(zbuf, acc.at[pl.ds(s * RPT + i * ZR, ZR)])
        return carry

    lax.fori_loop(0, RPT // ZR, body, 0)

    @pl.when(s == NS - 1)
    def _():
        pltpu.sync_copy(zbuf.at[pl.ds(0, TAIL)], acc.at[pl.ds(NS * RPT, TAIL)])


def _copy_out(acc, out_hbm, c, s):
    pltpu.sync_copy(acc.at[pl.ds(s * RPT, RPT)],
                    out_hbm.at[c, pl.ds(s * RPT, RPT)])

    @pl.when(s == NS - 1)
    def _():
        pltpu.sync_copy(acc.at[pl.ds(NS * RPT, TAIL)],
                        out_hbm.at[c, pl.ds(NS * RPT, TAIL)])


DW = 16


@functools.partial(
    pl.kernel,
    out_type=jax.ShapeDtypeStruct((NC, N, DW), jnp.float32),
    mesh=_mesh,
    scratch_types=[
        pltpu.VMEM((NCHUNK, K), jnp.int32),
        pltpu.VMEM((K, DW), jnp.float32),
        pltpu.VMEM((ZR, DW), jnp.float32),
        pltpu.VMEM_SHARED((N, DW), jnp.float32),
    ],
)
def _deg_kernel(dst_hbm, out_hbm, dst_v, ones_v, zbuf, acc):
    c = lax.axis_index("c")
    s = lax.axis_index("s")
    wid = s * NC + c

    def fill(buf, rows, value):
        vec = jnp.full((16,), value, jnp.float32)

        def body(i, carry):
            buf[i, pl.ds(0, 16)] = vec
            return carry

        lax.fori_loop(0, rows, body, 0)

    fill(zbuf, ZR, 0.0)

    def zbody(i, carry):
        pltpu.sync_copy(zbuf, acc.at[pl.ds(s * RPT + i * ZR, ZR)])
        return carry

    lax.fori_loop(0, RPT // ZR, zbody, 0)

    @pl.when(s == NS - 1)
    def _():
        pltpu.sync_copy(zbuf.at[pl.ds(0, TAIL)], acc.at[pl.ds(NS * RPT, TAIL)])

    fill(ones_v, K, 1.0)
    pltpu.sync_copy(dst_hbm.at[wid], dst_v)
    plsc.subcore_barrier()

    def body(j, carry):
        pltpu.sync_copy(ones_v, acc.at[dst_v.at[j]], add=True)
        return carry

    lax.fori_loop(0, NCHUNK, body, 0)
    plsc.subcore_barrier()
    pltpu.sync_copy(acc.at[pl.ds(s * RPT, RPT)],
                    out_hbm.at[c, pl.ds(s * RPT, RPT)])

    @pl.when(s == NS - 1)
    def _():
        pltpu.sync_copy(acc.at[pl.ds(NS * RPT, TAIL)],
                        out_hbm.at[c, pl.ds(NS * RPT, TAIL)])


@functools.partial(
    pl.kernel,
    out_type=jax.ShapeDtypeStruct((NC, N, D), jnp.float32),
    mesh=_mesh,
    scratch_types=[
        pltpu.VMEM((4, K), jnp.int32),
        pltpu.VMEM((4, K), jnp.int32),
        pltpu.VMEM((K, D), jnp.float32),
        pltpu.VMEM((K, D), jnp.float32),
        pltpu.VMEM((ZR, D), jnp.float32),
        pltpu.VMEM_SHARED((N, D), jnp.float32),
        pltpu.SemaphoreType.DMA,
        pltpu.SemaphoreType.DMA,
        pltpu.SemaphoreType.DMA,
        pltpu.SemaphoreType.DMA,
        pltpu.SemaphoreType.DMA,
        pltpu.SemaphoreType.DMA,
    ],
)
def _edge_kernel(y_hbm, src_hbm, dst_hbm, out_hbm,
                 sidx, didx, buf0, buf1, zbuf, acc,
                 sem0, sem1, isem0, isem1, isem2, isem3):
    c = lax.axis_index("c")
    s = lax.axis_index("s")
    wid = s * NC + c
    bufs = (buf0, buf1)
    sems = (sem0, sem1)
    isems = (isem0, isem1, isem2, isem3)

    def load_idx(chunk, row):
        pltpu.async_copy(src_hbm.at[wid, chunk], sidx.at[row], isems[row])
        pltpu.async_copy(dst_hbm.at[wid, chunk], didx.at[row], isems[row])

    def wait_idx(chunk, row):
        pltpu.make_async_copy(src_hbm.at[wid, chunk], sidx.at[row],
                              isems[row]).wait()
        pltpu.make_async_copy(dst_hbm.at[wid, chunk], didx.at[row],
                              isems[row]).wait()

    _zero_acc(zbuf, acc, s)
    for r in range(4):
        load_idx(r, r)
    plsc.subcore_barrier()
    for r in range(2):
        wait_idx(r, r)
        pltpu.async_copy(y_hbm.at[sidx.at[r]], bufs[r], sems[r])

    def body(jj, carry):
        for b in range(4):
            j = 4 * jj + b
            p = b % 2
            pltpu.make_async_copy(y_hbm.at[sidx.at[b]], bufs[p],
                                  sems[p]).wait()
            pltpu.sync_copy(bufs[p], acc.at[didx.at[b]], add=True)

            @pl.when(j + 4 < NCHUNK)
            def _():
                load_idx(j + 4, b)

            @pl.when(j + 2 < NCHUNK)
            def _():
                wait_idx(j + 2, (b + 2) % 4)
                pltpu.async_copy(y_hbm.at[sidx.at[(b + 2) % 4]], bufs[p],
                                 sems[p])
        return carry

    lax.fori_loop(0, NCHUNK // 4, body, 0)
    plsc.subcore_barrier()
    _copy_out(acc, out_hbm, c, s)


R = 1000


def _dis_block(degp_ref):
    deg = degp_ref[0, :, 0:1] + degp_ref[1, :, 0:1] + 1.0
    return lax.rsqrt(deg)


def _tc1_body(degp_ref, x_ref, w_ref, y_ref):
    y_ref[...] = _dis_block(degp_ref) * jnp.dot(
        x_ref[...], w_ref[...], preferred_element_type=jnp.float32)


def _tc2_body(degp_ref, aggp_ref, y1_ref, b1_ref, w_ref, y2_ref):
    dis = _dis_block(degp_ref)
    agg = aggp_ref[0] + aggp_ref[1] + y1_ref[...]
    h = jnp.maximum(dis * agg + b1_ref[...], 0.0)
    y2_ref[...] = dis * jnp.dot(h, w_ref[...],
                                preferred_element_type=jnp.float32)


def _tc3_body(degp_ref, aggp_ref, y2_ref, b2_ref, z_ref):
    z_ref[...] = (_dis_block(degp_ref)
                  * (aggp_ref[0] + aggp_ref[1] + y2_ref[...]) + b2_ref[...])


_deg_spec = pl.BlockSpec((NC, R, DW), lambda i: (0, i, 0))
_part_spec = pl.BlockSpec((NC, R, D), lambda i: (0, i, 0))
_row_spec = pl.BlockSpec((R, D), lambda i: (i, 0))
_mat_spec = pl.BlockSpec((D, D), lambda i: (0, 0))
_vec_spec = pl.BlockSpec((1, D), lambda i: (0, 0))
_rows = jax.ShapeDtypeStruct((N, D), jnp.float32)

_tc1 = pl.pallas_call(
    _tc1_body, grid=(N // R,),
    in_specs=[_deg_spec, _row_spec, _mat_spec],
    out_specs=_row_spec,
    out_shape=_rows,
)

_tc2 = pl.pallas_call(
    _tc2_body, grid=(N // R,),
    in_specs=[_deg_spec, _part_spec, _row_spec, _vec_spec, _mat_spec],
    out_specs=_row_spec,
    out_shape=_rows,
)

_tc3 = pl.pallas_call(
    _tc3_body, grid=(N // R,),
    in_specs=[_deg_spec, _part_spec, _row_spec, _vec_spec],
    out_specs=_row_spec,
    out_shape=_rows,
)


def kernel(x, edge_index, W1, b1, W2, b2):
    src3 = edge_index[0].reshape(NW, NCHUNK, K)
    dst3 = edge_index[1].reshape(NW, NCHUNK, K)
    degp = _deg_kernel(dst3)
    y1 = _tc1(degp, x, W1)
    agg1 = _edge_kernel(y1, src3, dst3)
    y2 = _tc2(degp, agg1, y1, b1.reshape(1, D), W2)
    agg2 = _edge_kernel(y2, src3, dst3)
    z = _tc3(degp, agg2, y2, b2.reshape(1, D))
    return z


# speedup vs baseline: 32.5480x; 1.0323x over previous
import functools

import jax
import jax.numpy as jnp
from jax import lax
from jax.experimental import pallas as pl
from jax.experimental.pallas import tpu as pltpu
from jax.experimental.pallas import tpu_sc as plsc

N = 10000
E = 320000
D = 128

NC = 2
NS = 16
NW = NC * NS
EPW = E // NW
K = 125
NCHUNK = EPW // K
RPT = 624
ZR = 16
TAIL = N - NS * RPT

_mesh = plsc.VectorSubcoreMesh(core_axis_name="c", subcore_axis_name="s")


def _fill(buf, rows, value):
    vec = jnp.full((16,), value, jnp.float32)

    def body(i, carry):
        buf[i // (D // 16), pl.ds((i % (D // 16)) * 16, 16)] = vec
        return carry

    lax.fori_loop(0, rows * (D // 16), body, 0)


def _zero_acc(zbuf, acc, s):
    _fill(zbuf, ZR, 0.0)

    def body(i, carry):
        pltpu.sync_copy(zbuf, acc.at[pl.ds(s * RPT + i * ZR, ZR)])
        return carry

    lax.fori_loop(0, RPT // ZR, body, 0)

    @pl.when(s == NS - 1)
    def _():
        pltpu.sync_copy(zbuf.at[pl.ds(0, TAIL)], acc.at[pl.ds(NS * RPT, TAIL)])


def _copy_out(acc, out_hbm, c, s):
    pltpu.sync_copy(acc.at[pl.ds(s * RPT, RPT)],
                    out_hbm.at[c, pl.ds(s * RPT, RPT)])

    @pl.when(s == NS - 1)
    def _():
        pltpu.sync_copy(acc.at[pl.ds(NS * RPT, TAIL)],
                        out_hbm.at[c, pl.ds(NS * RPT, TAIL)])


DW = 16


@functools.partial(
    pl.kernel,
    out_type=jax.ShapeDtypeStruct((NC, N, DW), jnp.float32),
    mesh=_mesh,
    scratch_types=[
        pltpu.VMEM((NCHUNK, K), jnp.int32),
        pltpu.VMEM((K, DW), jnp.float32),
        pltpu.VMEM((ZR, DW), jnp.float32),
        pltpu.VMEM_SHARED((N, DW), jnp.float32),
    ],
)
def _deg_kernel(dst_hbm, out_hbm, dst_v, ones_v, zbuf, acc):
    c = lax.axis_index("c")
    s = lax.axis_index("s")
    wid = s * NC + c

    def fill(buf, rows, value):
        vec = jnp.full((16,), value, jnp.float32)

        def body(i, carry):
            buf[i, pl.ds(0, 16)] = vec
            return carry

        lax.fori_loop(0, rows, body, 0)

    fill(zbuf, ZR, 0.0)

    def zbody(i, carry):
        pltpu.sync_copy(zbuf, acc.at[pl.ds(s * RPT + i * ZR, ZR)])
        return carry

    lax.fori_loop(0, RPT // ZR, zbody, 0)

    @pl.when(s == NS - 1)
    def _():
        pltpu.sync_copy(zbuf.at[pl.ds(0, TAIL)], acc.at[pl.ds(NS * RPT, TAIL)])

    fill(ones_v, K, 1.0)
    pltpu.sync_copy(dst_hbm.at[wid], dst_v)
    plsc.subcore_barrier()

    def body(j, carry):
        pltpu.sync_copy(ones_v, acc.at[dst_v.at[j]], add=True)
        return carry

    lax.fori_loop(0, NCHUNK, body, 0)
    plsc.subcore_barrier()
    pltpu.sync_copy(acc.at[pl.ds(s * RPT, RPT)],
                    out_hbm.at[c, pl.ds(s * RPT, RPT)])

    @pl.when(s == NS - 1)
    def _():
        pltpu.sync_copy(acc.at[pl.ds(NS * RPT, TAIL)],
                        out_hbm.at[c, pl.ds(NS * RPT, TAIL)])


@functools.partial(
    pl.kernel,
    out_type=jax.ShapeDtypeStruct((NC, N, D), jnp.float32),
    mesh=_mesh,
    scratch_types=[
        pltpu.VMEM((4, K), jnp.int32),
        pltpu.VMEM((4, K), jnp.int32),
        pltpu.VMEM((K, D), jnp.float32),
        pltpu.VMEM((K, D), jnp.float32),
        pltpu.VMEM((ZR, D), jnp.float32),
        pltpu.VMEM_SHARED((N, D), jnp.float32),
        pltpu.SemaphoreType.DMA,
        pltpu.SemaphoreType.DMA,
        pltpu.SemaphoreType.DMA,
        pltpu.SemaphoreType.DMA,
        pltpu.SemaphoreType.DMA,
        pltpu.SemaphoreType.DMA,
    ],
)
def _edge_kernel(y_hbm, src_hbm, dst_hbm, out_hbm,
                 sidx, didx, buf0, buf1, zbuf, acc,
                 sem0, sem1, isem0, isem1, isem2, isem3):
    c = lax.axis_index("c")
    s = lax.axis_index("s")
    wid = s * NC + c
    bufs = (buf0, buf1)
    sems = (sem0, sem1)
    isems = (isem0, isem1, isem2, isem3)

    def load_idx(chunk, row):
        pltpu.async_copy(src_hbm.at[wid, chunk], sidx.at[row], isems[row])
        pltpu.async_copy(dst_hbm.at[wid, chunk], didx.at[row], isems[row])

    def wait_idx(chunk, row):
        pltpu.make_async_copy(src_hbm.at[wid, chunk], sidx.at[row],
                              isems[row]).wait()
        pltpu.make_async_copy(dst_hbm.at[wid, chunk], didx.at[row],
                              isems[row]).wait()

    _zero_acc(zbuf, acc, s)
    for r in range(4):
        load_idx(r, r)
    plsc.subcore_barrier()
    for r in range(2):
        wait_idx(r, r)
        pltpu.async_copy(y_hbm.at[sidx.at[r]], bufs[r], sems[r])

    def body(jj, carry):
        for b in range(4):
            j = 4 * jj + b
            p = b % 2
            pltpu.make_async_copy(y_hbm.at[sidx.at[b]], bufs[p],
                                  sems[p]).wait()
            pltpu.sync_copy(bufs[p], acc.at[didx.at[b]], add=True)

            @pl.when(j + 4 < NCHUNK)
            def _():
                load_idx(j + 4, b)

            @pl.when(j + 2 < NCHUNK)
            def _():
                wait_idx(j + 2, (b + 2) % 4)
                pltpu.async_copy(y_hbm.at[sidx.at[(b + 2) % 4]], bufs[p],
                                 sems[p])
        return carry

    lax.fori_loop(0, NCHUNK // 4, body, 0)
    plsc.subcore_barrier()
    _copy_out(acc, out_hbm, c, s)


R = 1000


def _dis_block(degp_ref):
    deg = degp_ref[0, :, 0:1] + degp_ref[1, :, 0:1] + 1.0
    return lax.rsqrt(deg)


def _tc1_body(degp_ref, x_ref, w_ref, y_ref):
    y_ref[...] = _dis_block(degp_ref) * jnp.dot(
        x_ref[...], w_ref[...], preferred_element_type=jnp.float32)


def _tc2_body(degp_ref, aggp_ref, y1_ref, b1_ref, w_ref, y2_ref):
    dis = _dis_block(degp_ref)
    agg = aggp_ref[0] + aggp_ref[1] + y1_ref[...]
    h = jnp.maximum(dis * agg + b1_ref[...], 0.0)
    y2_ref[...] = dis * jnp.dot(h, w_ref[...],
                                preferred_element_type=jnp.float32)


def _tc3_body(degp_ref, aggp_ref, y2_ref, b2_ref, z_ref):
    z_ref[...] = (_dis_block(degp_ref)
                  * (aggp_ref[0] + aggp_ref[1] + y2_ref[...]) + b2_ref[...])


_deg_spec = pl.BlockSpec((NC, R, DW), lambda i: (0, i, 0))
_part_spec = pl.BlockSpec((NC, R, D), lambda i: (0, i, 0))
_row_spec = pl.BlockSpec((R, D), lambda i: (i, 0))
_mat_spec = pl.BlockSpec((D, D), lambda i: (0, 0))
_vec_spec = pl.BlockSpec((1, D), lambda i: (0, 0))
_rows = jax.ShapeDtypeStruct((N, D), jnp.float32)

_tc1 = pl.pallas_call(
    _tc1_body, grid=(N // R,),
    in_specs=[_deg_spec, _row_spec, _mat_spec],
    out_specs=_row_spec,
    out_shape=_rows,
)

_tc2 = pl.pallas_call(
    _tc2_body, grid=(N // R,),
    in_specs=[_deg_spec, _part_spec, _row_spec, _vec_spec, _mat_spec],
    out_specs=_row_spec,
    out_shape=_rows,
)

_tc3 = pl.pallas_call(
    _tc3_body, grid=(N // R,),
    in_specs=[_deg_spec, _part_spec, _row_spec, _vec_spec],
    out_specs=_row_spec,
    out_shape=_rows,
)


def kernel(x, edge_index, W1, b1, W2, b2):
    src3 = edge_index[0].reshape(NW, NCHUNK, K)
    dst3 = edge_index[1].reshape(NW, NCHUNK, K)
    degp = _deg_kernel(dst3)
    y1 = _tc1(degp, x, W1)
    agg1 = _edge_kernel(y1, src3, dst3)
    y2 = _tc2(degp, agg1, y1, b1.reshape(1, D), W2)
    agg2 = _edge_kernel(y2, src3, dst3)
    z = _tc3(degp, agg2, y2, b2.reshape(1, D))
    return z


# speedup vs baseline: 33.1645x; 1.0189x over previous
import functools

import jax
import jax.numpy as jnp
from jax import lax
from jax.experimental import pallas as pl
from jax.experimental.pallas import tpu as pltpu
from jax.experimental.pallas import tpu_sc as plsc

N = 10000
E = 320000
D = 128

NC = 2
NS = 16
NW = NC * NS
EPW = E // NW
K = 125
NCHUNK = EPW // K
RPT = 624
ZR = 16
TAIL = N - NS * RPT

_mesh = plsc.VectorSubcoreMesh(core_axis_name="c", subcore_axis_name="s")


def _fill(buf, rows, value):
    vec = jnp.full((16,), value, jnp.float32)

    def body(i, carry):
        buf[i // (D // 16), pl.ds((i % (D // 16)) * 16, 16)] = vec
        return carry

    lax.fori_loop(0, rows * (D // 16), body, 0)


def _zero_acc(zbuf, acc, s):
    _fill(zbuf, ZR, 0.0)

    def body(i, carry):
        pltpu.sync_copy(zbuf, acc.at[pl.ds(s * RPT + i * ZR, ZR)])
        return carry

    lax.fori_loop(0, RPT // ZR, body, 0)

    @pl.when(s == NS - 1)
    def _():
        pltpu.sync_copy(zbuf.at[pl.ds(0, TAIL)], acc.at[pl.ds(NS * RPT, TAIL)])


def _copy_out(acc, out_hbm, c, s):
    pltpu.sync_copy(acc.at[pl.ds(s * RPT, RPT)],
                    out_hbm.at[c, pl.ds(s * RPT, RPT)])

    @pl.when(s == NS - 1)
    def _():
        pltpu.sync_copy(acc.at[pl.ds(NS * RPT, TAIL)],
                        out_hbm.at[c, pl.ds(NS * RPT, TAIL)])


DW = 16
DK = 100
DNCHUNK = EPW // DK


@functools.partial(
    pl.kernel,
    out_type=jax.ShapeDtypeStruct((NC, N, DW), jnp.float32),
    mesh=_mesh,
    scratch_types=[
        pltpu.VMEM((DNCHUNK, DK), jnp.int32),
        pltpu.VMEM((DK, DW), jnp.float32),
        pltpu.VMEM((ZR, DW), jnp.float32),
        pltpu.VMEM_SHARED((N, DW), jnp.float32),
    ],
)
def _deg_kernel(dst_hbm, out_hbm, dst_v, ones_v, zbuf, acc):
    c = lax.axis_index("c")
    s = lax.axis_index("s")
    wid = s * NC + c

    def fill(buf, rows, value):
        vec = jnp.full((16,), value, jnp.float32)

        def body(i, carry):
            buf[i, pl.ds(0, 16)] = vec
            return carry

        lax.fori_loop(0, rows, body, 0)

    fill(zbuf, ZR, 0.0)

    def zbody(i, carry):
        pltpu.sync_copy(zbuf, acc.at[pl.ds(s * RPT + i * ZR, ZR)])
        return carry

    lax.fori_loop(0, RPT // ZR, zbody, 0)

    @pl.when(s == NS - 1)
    def _():
        pltpu.sync_copy(zbuf.at[pl.ds(0, TAIL)], acc.at[pl.ds(NS * RPT, TAIL)])

    fill(ones_v, DK, 1.0)
    pltpu.sync_copy(dst_hbm.at[wid], dst_v)
    plsc.subcore_barrier()

    def body(j, carry):
        pltpu.sync_copy(ones_v, acc.at[dst_v.at[j]], add=True)
        return carry

    lax.fori_loop(0, DNCHUNK, body, 0)
    plsc.subcore_barrier()
    pltpu.sync_copy(acc.at[pl.ds(s * RPT, RPT)],
                    out_hbm.at[c, pl.ds(s * RPT, RPT)])

    @pl.when(s == NS - 1)
    def _():
        pltpu.sync_copy(acc.at[pl.ds(NS * RPT, TAIL)],
                        out_hbm.at[c, pl.ds(NS * RPT, TAIL)])


@functools.partial(
    pl.kernel,
    out_type=jax.ShapeDtypeStruct((NC, N, D), jnp.float32),
    mesh=_mesh,
    scratch_types=[
        pltpu.VMEM((4, K), jnp.int32),
        pltpu.VMEM((4, K), jnp.int32),
        pltpu.VMEM((K, D), jnp.float32),
        pltpu.VMEM((K, D), jnp.float32),
        pltpu.VMEM((ZR, D), jnp.float32),
        pltpu.VMEM_SHARED((N, D), jnp.float32),
        pltpu.SemaphoreType.DMA,
        pltpu.SemaphoreType.DMA,
        pltpu.SemaphoreType.DMA,
        pltpu.SemaphoreType.DMA,
        pltpu.SemaphoreType.DMA,
        pltpu.SemaphoreType.DMA,
    ],
)
def _edge_kernel(y_hbm, src_hbm, dst_hbm, out_hbm,
                 sidx, didx, buf0, buf1, zbuf, acc,
                 sem0, sem1, isem0, isem1, isem2, isem3):
    c = lax.axis_index("c")
    s = lax.axis_index("s")
    wid = s * NC + c
    bufs = (buf0, buf1)
    sems = (sem0, sem1)
    isems = (isem0, isem1, isem2, isem3)

    def load_idx(chunk, row):
        pltpu.async_copy(src_hbm.at[wid, chunk], sidx.at[row], isems[row])
        pltpu.async_copy(dst_hbm.at[wid, chunk], didx.at[row], isems[row])

    def wait_idx(chunk, row):
        pltpu.make_async_copy(src_hbm.at[wid, chunk], sidx.at[row],
                              isems[row]).wait()
        pltpu.make_async_copy(dst_hbm.at[wid, chunk], didx.at[row],
                              isems[row]).wait()

    _zero_acc(zbuf, acc, s)
    for r in range(4):
        load_idx(r, r)
    plsc.subcore_barrier()
    for r in range(2):
        wait_idx(r, r)
        pltpu.async_copy(y_hbm.at[sidx.at[r]], bufs[r], sems[r])

    def body(jj, carry):
        for b in range(4):
            j = 4 * jj + b
            p = b % 2
            pltpu.make_async_copy(y_hbm.at[sidx.at[b]], bufs[p],
                                  sems[p]).wait()
            pltpu.sync_copy(bufs[p], acc.at[didx.at[b]], add=True)

            @pl.when(j + 4 < NCHUNK)
            def _():
                load_idx(j + 4, b)

            @pl.when(j + 2 < NCHUNK)
            def _():
                wait_idx(j + 2, (b + 2) % 4)
                pltpu.async_copy(y_hbm.at[sidx.at[(b + 2) % 4]], bufs[p],
                                 sems[p])
        return carry

    lax.fori_loop(0, NCHUNK // 4, body, 0)
    plsc.subcore_barrier()
    _copy_out(acc, out_hbm, c, s)


R = 2000


def _dis_block(degp_ref):
    deg = degp_ref[0, :, 0:1] + degp_ref[1, :, 0:1] + 1.0
    return lax.rsqrt(deg)


def _tc1_body(degp_ref, x_ref, w_ref, y_ref):
    y_ref[...] = _dis_block(degp_ref) * jnp.dot(
        x_ref[...], w_ref[...], preferred_element_type=jnp.float32)


def _tc2_body(degp_ref, aggp_ref, y1_ref, b1_ref, w_ref, y2_ref):
    dis = _dis_block(degp_ref)
    agg = aggp_ref[0] + aggp_ref[1] + y1_ref[...]
    h = jnp.maximum(dis * agg + b1_ref[...], 0.0)
    y2_ref[...] = dis * jnp.dot(h, w_ref[...],
                                preferred_element_type=jnp.float32)


def _tc3_body(degp_ref, aggp_ref, y2_ref, b2_ref, z_ref):
    z_ref[...] = (_dis_block(degp_ref)
                  * (aggp_ref[0] + aggp_ref[1] + y2_ref[...]) + b2_ref[...])


_deg_spec = pl.BlockSpec((NC, R, DW), lambda i: (0, i, 0))
_part_spec = pl.BlockSpec((NC, R, D), lambda i: (0, i, 0))
_row_spec = pl.BlockSpec((R, D), lambda i: (i, 0))
_mat_spec = pl.BlockSpec((D, D), lambda i: (0, 0))
_vec_spec = pl.BlockSpec((1, D), lambda i: (0, 0))
_rows = jax.ShapeDtypeStruct((N, D), jnp.float32)

_tc1 = pl.pallas_call(
    _tc1_body, grid=(N // R,),
    in_specs=[_deg_spec, _row_spec, _mat_spec],
    out_specs=_row_spec,
    out_shape=_rows,
)

_tc2 = pl.pallas_call(
    _tc2_body, grid=(N // R,),
    in_specs=[_deg_spec, _part_spec, _row_spec, _vec_spec, _mat_spec],
    out_specs=_row_spec,
    out_shape=_rows,
)

_tc3 = pl.pallas_call(
    _tc3_body, grid=(N // R,),
    in_specs=[_deg_spec, _part_spec, _row_spec, _vec_spec],
    out_specs=_row_spec,
    out_shape=_rows,
)


def kernel(x, edge_index, W1, b1, W2, b2):
    src3 = edge_index[0].reshape(NW, NCHUNK, K)
    dst3 = edge_index[1].reshape(NW, NCHUNK, K)
    degp = _deg_kernel(edge_index[1].reshape(NW, DNCHUNK, DK))
    y1 = _tc1(degp, x, W1)
    agg1 = _edge_kernel(y1, src3, dst3)
    y2 = _tc2(degp, agg1, y1, b1.reshape(1, D), W2)
    agg2 = _edge_kernel(y2, src3, dst3)
    z = _tc3(degp, agg2, y2, b2.reshape(1, D))
    return z


# speedup vs baseline: 33.1763x; 1.0004x over previous
import functools

import jax
import jax.numpy as jnp
from jax import lax
from jax.experimental import pallas as pl
from jax.experimental.pallas import tpu as pltpu
from jax.experimental.pallas import tpu_sc as plsc

N = 10000
E = 320000
D = 128

NC = 2
NS = 16
NW = NC * NS
EPW = E // NW
K = 125
NCHUNK = EPW // K
RPT = 624
ZR = 16
TAIL = N - NS * RPT

_mesh = plsc.VectorSubcoreMesh(core_axis_name="c", subcore_axis_name="s")


def _fill(buf, rows, value):
    vec = jnp.full((16,), value, jnp.float32)

    def body(i, carry):
        buf[i // (D // 16), pl.ds((i % (D // 16)) * 16, 16)] = vec
        return carry

    lax.fori_loop(0, rows * (D // 16), body, 0)


def _zero_acc(zbuf, acc, s):
    _fill(zbuf, ZR, 0.0)

    def body(i, carry):
        pltpu.sync_copy(zbuf, acc.at[pl.ds(s * RPT + i * ZR, ZR)])
        return carry

    lax.fori_loop(0, RPT // ZR, body, 0)

    @pl.when(s == NS - 1)
    def _():
        pltpu.sync_copy(zbuf.at[pl.ds(0, TAIL)], acc.at[pl.ds(NS * RPT, TAIL)])


def _copy_out(acc, out_hbm, c, s):
    pltpu.sync_copy(acc.at[pl.ds(s * RPT, RPT)],
                    out_hbm.at[c, pl.ds(s * RPT, RPT)])

    @pl.when(s == NS - 1)
    def _():
        pltpu.sync_copy(acc.at[pl.ds(NS * RPT, TAIL)],
                        out_hbm.at[c, pl.ds(NS * RPT, TAIL)])


DW = 16
DK = 100
DNCHUNK = EPW // DK


@functools.partial(
    pl.kernel,
    out_type=jax.ShapeDtypeStruct((NC, N, DW), jnp.float32),
    mesh=_mesh,
    scratch_types=[
        pltpu.VMEM((DNCHUNK, DK), jnp.int32),
        pltpu.VMEM((DK, DW), jnp.float32),
        pltpu.VMEM((ZR, DW), jnp.float32),
        pltpu.VMEM_SHARED((N, DW), jnp.float32),
    ],
)
def _deg_kernel(dst_hbm, out_hbm, dst_v, ones_v, zbuf, acc):
    c = lax.axis_index("c")
    s = lax.axis_index("s")
    wid = s * NC + c

    def fill(buf, rows, value):
        vec = jnp.full((16,), value, jnp.float32)

        def body(i, carry):
            buf[i, pl.ds(0, 16)] = vec
            return carry

        lax.fori_loop(0, rows, body, 0)

    fill(zbuf, ZR, 0.0)

    def zbody(i, carry):
        pltpu.sync_copy(zbuf, acc.at[pl.ds(s * RPT + i * ZR, ZR)])
        return carry

    lax.fori_loop(0, RPT // ZR, zbody, 0)

    @pl.when(s == NS - 1)
    def _():
        pltpu.sync_copy(zbuf.at[pl.ds(0, TAIL)], acc.at[pl.ds(NS * RPT, TAIL)])

    fill(ones_v, DK, 1.0)
    pltpu.sync_copy(dst_hbm.at[wid], dst_v)
    plsc.subcore_barrier()

    def body(j, carry):
        pltpu.sync_copy(ones_v, acc.at[dst_v.at[j]], add=True)
        return carry

    lax.fori_loop(0, DNCHUNK, body, 0)
    plsc.subcore_barrier()
    pltpu.sync_copy(acc.at[pl.ds(s * RPT, RPT)],
                    out_hbm.at[c, pl.ds(s * RPT, RPT)])

    @pl.when(s == NS - 1)
    def _():
        pltpu.sync_copy(acc.at[pl.ds(NS * RPT, TAIL)],
                        out_hbm.at[c, pl.ds(NS * RPT, TAIL)])


@functools.partial(
    pl.kernel,
    out_type=jax.ShapeDtypeStruct((NC, N, D), jnp.float32),
    mesh=_mesh,
    scratch_types=[
        pltpu.VMEM((4, K), jnp.int32),
        pltpu.VMEM((4, K), jnp.int32),
        pltpu.VMEM((K, D), jnp.float32),
        pltpu.VMEM((K, D), jnp.float32),
        pltpu.VMEM((ZR, D), jnp.float32),
        pltpu.VMEM_SHARED((N, D), jnp.float32),
        pltpu.SemaphoreType.DMA,
        pltpu.SemaphoreType.DMA,
        pltpu.SemaphoreType.DMA,
        pltpu.SemaphoreType.DMA,
        pltpu.SemaphoreType.DMA,
        pltpu.SemaphoreType.DMA,
    ],
)
def _edge_kernel(y_hbm, src_hbm, dst_hbm, out_hbm,
                 sidx, didx, buf0, buf1, zbuf, acc,
                 gsem0, gsem1, isem0, isem1, isem2, isem3):
    c = lax.axis_index("c")
    s = lax.axis_index("s")
    wid = s * NC + c
    bufs = (buf0, buf1)
    gsems = (gsem0, gsem1)
    isems = (isem0, isem1, isem2, isem3)

    def load_idx(chunk, row):
        pltpu.async_copy(src_hbm.at[wid, chunk], sidx.at[row], isems[row])
        pltpu.async_copy(dst_hbm.at[wid, chunk], didx.at[row], isems[row])

    def wait_idx(chunk, row):
        pltpu.make_async_copy(src_hbm.at[wid, chunk], sidx.at[row],
                              isems[row]).wait()
        pltpu.make_async_copy(dst_hbm.at[wid, chunk], didx.at[row],
                              isems[row]).wait()

    _zero_acc(zbuf, acc, s)
    for r in range(4):
        load_idx(r, r)
    plsc.subcore_barrier()
    for r in range(2):
        wait_idx(r, r)
        pltpu.async_copy(y_hbm.at[sidx.at[r]], bufs[r], gsems[r])

    def body(jj, carry):
        for b in range(4):
            j = 4 * jj + b
            p = b % 2
            pltpu.make_async_copy(y_hbm.at[sidx.at[b]], bufs[p],
                                  gsems[p]).wait()
            pltpu.sync_copy(bufs[p], acc.at[didx.at[b]], add=True)

            @pl.when(j + 4 < NCHUNK)
            def _():
                load_idx(j + 4, b)

            @pl.when(j + 2 < NCHUNK)
            def _():
                wait_idx(j + 2, (b + 2) % 4)
                pltpu.async_copy(y_hbm.at[sidx.at[(b + 2) % 4]], bufs[p],
                                 gsems[p])
        return carry

    lax.fori_loop(0, NCHUNK // 4, body, 0)
    plsc.subcore_barrier()
    _copy_out(acc, out_hbm, c, s)


R = 2000


def _dis_block(degp_ref):
    deg = degp_ref[0, :, 0:1] + degp_ref[1, :, 0:1] + 1.0
    return lax.rsqrt(deg)


def _f32(ref):
    return ref[...].astype(jnp.float32)


def _tcmm_body(x_ref, w_ref, xw_ref):
    xw_ref[...] = jnp.dot(x_ref[...], w_ref[...],
                          preferred_element_type=jnp.float32)


def _tc1_body(degp_ref, xw_ref, y_ref):
    y_ref[...] = _dis_block(degp_ref) * xw_ref[...]


def _tc2_body(degp_ref, aggp_ref, y1_ref, b1_ref, w_ref, y2_ref):
    dis = _dis_block(degp_ref)
    agg = aggp_ref[0] + aggp_ref[1] + y1_ref[...]
    h = jnp.maximum(dis * agg + b1_ref[...], 0.0)
    y2_ref[...] = dis * jnp.dot(h, w_ref[...],
                                preferred_element_type=jnp.float32)


def _tc3_body(degp_ref, aggp_ref, y2_ref, b2_ref, z_ref):
    agg = aggp_ref[0] + aggp_ref[1] + y2_ref[...]
    z_ref[...] = _dis_block(degp_ref) * agg + b2_ref[...]


_deg_spec = pl.BlockSpec((NC, R, DW), lambda i: (0, i, 0))
_part_spec = pl.BlockSpec((NC, R, D), lambda i: (0, i, 0))
_row_spec = pl.BlockSpec((R, D), lambda i: (i, 0))
_mat_spec = pl.BlockSpec((D, D), lambda i: (0, 0))
_vec_spec = pl.BlockSpec((1, D), lambda i: (0, 0))
_rows = jax.ShapeDtypeStruct((N, D), jnp.float32)

_tcmm = pl.pallas_call(
    _tcmm_body, grid=(N // R,),
    in_specs=[_row_spec, _mat_spec],
    out_specs=_row_spec,
    out_shape=_rows,
)

_tc1 = pl.pallas_call(
    _tc1_body, grid=(N // R,),
    in_specs=[_deg_spec, _row_spec],
    out_specs=_row_spec,
    out_shape=_rows,
)

_tc2 = pl.pallas_call(
    _tc2_body, grid=(N // R,),
    in_specs=[_deg_spec, _part_spec, _row_spec, _vec_spec, _mat_spec],
    out_specs=_row_spec,
    out_shape=_rows,
)

_tc3 = pl.pallas_call(
    _tc3_body, grid=(N // R,),
    in_specs=[_deg_spec, _part_spec, _row_spec, _vec_spec],
    out_specs=_row_spec,
    out_shape=_rows,
)


def kernel(x, edge_index, W1, b1, W2, b2):
    src3 = edge_index[0].reshape(NW, NCHUNK, K)
    dst3 = edge_index[1].reshape(NW, NCHUNK, K)
    degp = _deg_kernel(edge_index[1].reshape(NW, DNCHUNK, DK))
    xw1 = _tcmm(x, W1)
    y1 = _tc1(degp, xw1)
    agg1 = _edge_kernel(y1, src3, dst3)
    y2 = _tc2(degp, agg1, y1, b1.reshape(1, D), W2)
    agg2 = _edge_kernel(y2, src3, dst3)
    z = _tc3(degp, agg2, y2, b2.reshape(1, D))
    return z
